# baseline (speedup 1.0000x reference)
import jax
import jax.numpy as jnp
from jax import lax
from jax.experimental import pallas as pl
from jax.experimental.pallas import tpu as pltpu
from jax.experimental.pallas import tpu_sc as plsc

EMBED_DIM = 11
NUM_ROWS = 1_000_000
BATCH = 16384


def _make_sc_gather(num_workers: int, w_per_w: int):
    mesh = plsc.VectorSubcoreMesh(core_axis_name="c", subcore_axis_name="s")

    @pl.kernel(
        out_type=jax.ShapeDtypeStruct((BATCH * EMBED_DIM,), jnp.float32),
        mesh=mesh,
        scratch_types=[
            pltpu.VMEM((w_per_w,), jnp.int32),
            pltpu.VMEM((w_per_w,), jnp.float32),
            pltpu.SemaphoreType.DMA,
        ],
        compiler_params=pltpu.CompilerParams(use_tc_tiling_on_sc=False),
    )
    def k(widx_hbm, table_hbm, out_hbm, idx_v, vals_v, sem):
        wid = lax.axis_index("s") * 2 + lax.axis_index("c")
        base = wid * w_per_w
        pltpu.sync_copy(widx_hbm.at[pl.ds(base, w_per_w)], idx_v)
        pltpu.async_copy(table_hbm.at[idx_v], vals_v, sem).wait()
        pltpu.sync_copy(vals_v, out_hbm.at[pl.ds(base, w_per_w)])

    return k


def kernel(device_num_tensor, table):
    info = plsc.get_sparse_core_info()
    num_workers = info.num_cores * info.num_subcores
    w_per_w = BATCH * EMBED_DIM // num_workers
    idx = device_num_tensor.astype(jnp.int32)
    widx = jnp.concatenate([idx + c * NUM_ROWS for c in range(EMBED_DIM)])
    flat_t = table.T.reshape(-1)
    out = _make_sc_gather(num_workers, w_per_w)(widx, flat_t)
    return out.reshape(EMBED_DIM, BATCH).T


# speedup vs baseline: 1.6997x; 1.6997x over previous
import jax
import jax.numpy as jnp
from jax import lax
from jax.experimental import pallas as pl
from jax.experimental.pallas import tpu as pltpu
from jax.experimental.pallas import tpu_sc as plsc

EMBED_DIM = 11
PAD_DIM = 16
BATCH = 16384
LANES = 16


def _make_sc_gather(num_workers: int, b_per_w: int):
    mesh = plsc.VectorSubcoreMesh(core_axis_name="c", subcore_axis_name="s")
    w_per_w = b_per_w * PAD_DIM
    n_chunks = b_per_w // LANES

    @pl.kernel(
        out_type=jax.ShapeDtypeStruct((BATCH * PAD_DIM,), jnp.float32),
        mesh=mesh,
        scratch_types=[
            pltpu.VMEM((b_per_w,), jnp.int32),
            pltpu.VMEM((w_per_w,), jnp.int32),
            pltpu.VMEM((w_per_w,), jnp.float32),
            pltpu.SemaphoreType.DMA,
        ],
        compiler_params=pltpu.CompilerParams(
            use_tc_tiling_on_sc=False, needs_layout_passes=False
        ),
    )
    def k(idx_hbm, table_hbm, out_hbm, idx_v, widx_v, vals_v, sem):
        wid = lax.axis_index("s") * 2 + lax.axis_index("c")
        pltpu.sync_copy(idx_hbm.at[pl.ds(wid * b_per_w, b_per_w)], idx_v)

        lane_iota = jax.lax.iota(jnp.int32, LANES)

        def expand(kk, carry):
            base = idx_v[pl.ds(kk * LANES, LANES)] * EMBED_DIM
            dst0 = kk * (LANES * PAD_DIM)
            for c in range(PAD_DIM):
                vals = base + min(c, EMBED_DIM - 1)
                plsc.store_scatter(widx_v, [dst0 + lane_iota * PAD_DIM + c], vals)
            return carry

        lax.fori_loop(0, n_chunks, expand, 0)

        pltpu.async_copy(table_hbm.at[widx_v], vals_v, sem).wait()
        pltpu.sync_copy(vals_v, out_hbm.at[pl.ds(wid * w_per_w, w_per_w)])

    return k


def kernel(device_num_tensor, table):
    info = plsc.get_sparse_core_info()
    num_workers = info.num_cores * info.num_subcores
    b_per_w = BATCH // num_workers
    idx = device_num_tensor.astype(jnp.int32)
    flat = table.reshape(-1)
    out = _make_sc_gather(num_workers, b_per_w)(idx, flat)
    return out.reshape(BATCH, PAD_DIM)[:, :EMBED_DIM]
